# Initial kernel scaffold; baseline (speedup 1.0000x reference)
import jax
import jax.numpy as jnp
from jax.experimental import pallas as pl


def kernel(var_node_features, con_node_features, edge_features, params, node_types, edge_index, edge_types, assoc_con, assoc_var):
    raise NotImplementedError("write your pallas kernel here")


# speedup vs baseline: 6.9572x; 6.9572x over previous
import functools

import jax
import jax.numpy as jnp
from jax import lax
from jax.experimental import pallas as pl
from jax.experimental.pallas import tpu as pltpu
from jax.experimental.pallas import tpu_sc as plsc

N = 10000
NP = 10240
D = 128
E = 320000
NC = 2
NS = 16
NW = NC * NS
K = 128
CPW = -(-E // (NW * K))
EP = NW * K * CPW
RPT = NP // NS
BR = 512
F32 = jnp.float32


def _embed_body(vf_ref, w1_ref, b1_ref, w2_ref, b2_ref, o_ref, *, extra_one):
    vf = vf_ref[...]
    h1 = jnp.maximum(vf * w1_ref[...] + b1_ref[...], 0.0)
    h2 = jnp.dot(h1, w2_ref[...], preferred_element_type=F32) + b2_ref[...]
    col = lax.broadcasted_iota(jnp.int32, h2.shape, 1)
    if extra_one:
        h2 = h2 + jnp.where(col == D - 2, vf, 0.0)
        h2 = h2 + jnp.where(col == D - 1, 1.0, 0.0)
    else:
        h2 = h2 + jnp.where(col == D - 1, vf, 0.0)
    o_ref[...] = h2


def _embed(vf, w1p, b1p, w2p, b2p, extra_one):
    rows = vf.shape[0]
    return pl.pallas_call(
        functools.partial(_embed_body, extra_one=extra_one),
        out_shape=jax.ShapeDtypeStruct((rows, D), F32),
    )(vf, w1p, b1p, w2p, b2p)


def _tables_body(x_ref, ef_ref, wcp_ref, wv_ref, w1_ref, b1_ref, w2p_ref,
                 b2_ref, o_ref):
    xb = x_ref[...]
    a1 = jnp.dot(xb, wv_ref[...], preferred_element_type=F32)
    sg = jax.nn.sigmoid(jnp.dot(xb, w1_ref[...], preferred_element_type=F32)
                        + b1_ref[...])
    va = (jnp.dot(sg, w2p_ref[...], preferred_element_type=F32)
          + b2_ref[...])[:, 0:1] * ef_ref[...]
    base = jnp.dot(xb, wcp_ref[...], preferred_element_type=F32)
    col = lax.broadcasted_iota(jnp.int32, base.shape, 1)
    a0 = base + jnp.where(col == D - 2, xb[:, D - 2:D - 1], 0.0)
    a0 = a0 + jnp.where(col == D - 1, va, 0.0)
    o_ref[0] = a0
    o_ref[1] = a1


def _tables(x, efp, wcp, wv, w1, b1, w2p, b2r):
    full = lambda s: pl.BlockSpec(s, lambda i: (0,) * len(s))
    return pl.pallas_call(
        _tables_body,
        grid=(NP // BR,),
        in_specs=[
            pl.BlockSpec((BR, D), lambda i: (i, 0)),
            pl.BlockSpec((BR, 1), lambda i: (i, 0)),
            full((D, D)), full((D, D)), full((D, D)), full((1, D)),
            full((D, D)), full((1, D)),
        ],
        out_specs=pl.BlockSpec((2, BR, D), lambda i: (0, i, 0)),
        out_shape=jax.ShapeDtypeStruct((2, NP, D), F32),
    )(x, efp, wcp, wv, w1, b1, w2p, b2r)


def _combine_body(p_ref, b_ref, o_ref):
    o_ref[...] = jnp.maximum(p_ref[0] + p_ref[1] + b_ref[...], 0.0)


def _combine(p, brow):
    return pl.pallas_call(
        _combine_body,
        grid=(NP // BR,),
        in_specs=[pl.BlockSpec((2, BR, D), lambda i: (0, i, 0)),
                  pl.BlockSpec((1, D), lambda i: (0, 0))],
        out_specs=pl.BlockSpec((BR, D), lambda i: (i, 0)),
        out_shape=jax.ShapeDtypeStruct((NP, D), F32),
    )(p, brow)


def _final_body(x0, x1, x2, x3, x4, w1, b1, w2, b2, w3, b3, w4p, b4, o_ref):
    h = jnp.dot(x0[...], w1[0], preferred_element_type=F32)
    h += jnp.dot(x1[...], w1[1], preferred_element_type=F32)
    h += jnp.dot(x2[...], w1[2], preferred_element_type=F32)
    h += jnp.dot(x3[...], w1[3], preferred_element_type=F32)
    h += jnp.dot(x4[...], w1[4], preferred_element_type=F32)
    h = jnp.maximum(h + b1[...], 0.0)
    h = jnp.maximum(jnp.dot(h, w2[...], preferred_element_type=F32) + b2[...], 0.0)
    h = jnp.maximum(jnp.dot(h, w3[...], preferred_element_type=F32) + b3[...], 0.0)
    o_ref[...] = jnp.dot(h, w4p[...], preferred_element_type=F32) + b4[...]


def _final(xs, w1r, b1r, w2, b2r, w3, b3r, w4p, b4r):
    BF = 600
    full = lambda s: pl.BlockSpec(s, lambda i: (0,) * len(s))
    return pl.pallas_call(
        _final_body,
        grid=(6000 // BF,),
        in_specs=[pl.BlockSpec((BF, D), lambda i: (i, 0))] * 5 + [
            full((5, D, D)), full((1, D)), full((D, D)), full((1, D)),
            full((D, D)), full((1, D)), full((D, D)), full((1, D)),
        ],
        out_specs=pl.BlockSpec((BF, D), lambda i: (i, 0)),
        out_shape=jax.ShapeDtypeStruct((6000, D), F32),
    )(*xs, w1r, b1r, w2, b2r, w3, b3r, w4p, b4r)


def _sc_body(t_hbm, idx_hbm, dst_hbm, z_hbm, out_hbm, idx_v, dst_v, buf, acc,
             sem):
    cid = lax.axis_index("c")
    sid = lax.axis_index("s")
    wid = cid * NS + sid
    pltpu.sync_copy(z_hbm, acc.at[pl.ds(sid * RPT, RPT)])
    pltpu.sync_copy(idx_hbm.at[wid], idx_v)
    pltpu.sync_copy(dst_hbm.at[wid], dst_v)
    plsc.subcore_barrier()

    def chunk(j, carry):
        pltpu.async_copy(t_hbm.at[idx_v.at[j]], buf, sem).wait()
        pltpu.sync_copy(buf, acc.at[dst_v.at[j]], add=True)
        return carry

    lax.fori_loop(0, CPW, chunk, 0, unroll=False)
    plsc.subcore_barrier()
    pltpu.sync_copy(acc.at[pl.ds(sid * RPT, RPT)],
                    out_hbm.at[cid, pl.ds(sid * RPT, RPT)])


def _sc_aggregate(t, idx, dst, zeros):
    call = pl.kernel(
        _sc_body,
        out_type=jax.ShapeDtypeStruct((NC, NP, D), F32),
        mesh=plsc.VectorSubcoreMesh(core_axis_name="c", subcore_axis_name="s",
                                    num_cores=NC, num_subcores=NS),
        scratch_types=[
            pltpu.VMEM((CPW, K), jnp.int32),
            pltpu.VMEM((CPW, K), jnp.int32),
            pltpu.VMEM((K, D), F32),
            pltpu.VMEM_SHARED((NP, D), F32),
            pltpu.SemaphoreType.DMA,
        ],
    )
    return call(t, idx, dst, zeros)


def _pad_w(w, r=D, c=D):
    return jnp.pad(w, ((0, r - w.shape[0]), (0, c - w.shape[1]))).astype(F32)


def _row(b, c=D):
    return jnp.pad(b.astype(F32), (0, c - b.shape[0])).reshape(1, c)


def kernel(var_node_features, con_node_features, edge_features, params,
           node_types, edge_index, edge_types, assoc_con, assoc_var):
    p = params
    nv = var_node_features.shape[0]

    src = edge_index[0].astype(jnp.int32)
    dst = edge_index[1].astype(jnp.int32)
    idx = edge_types.astype(jnp.int32) * NP + src
    idx = jnp.concatenate([idx, jnp.zeros((EP - E,), jnp.int32)])
    dst = jnp.concatenate([dst, jnp.full((EP - E,), N, jnp.int32)])
    idx = idx.reshape(NW, CPW, K)
    dst = dst.reshape(NW, CPW, K)
    efp = jnp.pad(edge_features.astype(F32), ((0, NP - N), (0, 0)))
    zeros = jnp.zeros((RPT, D), F32)

    vm, cm = p["var_mlp"], p["con_mlp"]
    nvar = _embed(var_node_features.astype(F32), _pad_w(vm["W1"], 1, D),
                  _row(vm["b1"]), _pad_w(vm["W2"]), _row(vm["b2"]), True)
    ncon = _embed(con_node_features.astype(F32), _pad_w(cm["W1"], 1, D),
                  _row(cm["b1"]), _pad_w(cm["W2"]), _row(cm["b2"]), False)
    x = jnp.concatenate(
        [nvar, ncon, jnp.zeros((NP - N, D), F32)], axis=0)

    xs = [x]
    for cp in p["convs"]:
        t = _tables(x, efp, _pad_w(cp["w_cons"]), cp["w_vars"].astype(F32),
                    cp["h2v_W1"].astype(F32), _row(cp["h2v_b1"]),
                    _pad_w(cp["h2v_W2"]),
                    jnp.full((1, D), cp["h2v_b2"][0], F32))
        part = _sc_aggregate(t.reshape(2 * NP, D), idx, dst, zeros)
        x = _combine(part, _row(cp["bias"]))
        xs.append(x)

    fc1, fc2, fc3, fc4 = p["fc1"], p["fc2"], p["fc3"], p["fc4"]
    out = _final(xs, fc1["W"].astype(F32).reshape(5, D, D), _row(fc1["b"]),
                 fc2["W"].astype(F32), _row(fc2["b"]),
                 fc3["W"].astype(F32), _row(fc3["b"]),
                 _pad_w(fc4["W"]), jnp.full((1, D), fc4["b"][0], F32))
    return out[:nv, 0]
